# baseline (speedup 1.0000x reference)
import functools

import jax
import jax.numpy as jnp
from jax import lax
from jax.experimental import pallas as pl
from jax.experimental.pallas import tpu as pltpu
from jax.experimental.pallas import tpu_sc as plsc

N_NODES = 50000
N_USERS = 10000
N_ENTITIES = 40000
D = 64
H = 32
CHUNK = 128
NSUB = 16
B = 4096
DR = 128

f32 = jnp.float32
i32 = jnp.int32


def _mesh():
    return plsc.VectorSubcoreMesh(core_axis_name="c", subcore_axis_name="s")


def _fill(ref, n_rows, value):
    k = ref.shape[1] // 16
    def body(i, _):
        for j in range(k):
            ref[i, pl.ds(16 * j, 16)] = jnp.full((16,), value, f32)
        return 0
    lax.fori_loop(0, n_rows, body, 0)


def _make_deg_kernel(n_pad_rows, n_blocks):
    rows_pt = n_pad_rows // NSUB
    nd = rows_pt // DR

    @functools.partial(
        pl.kernel, mesh=_mesh(),
        compiler_params=pltpu.CompilerParams(use_tc_tiling_on_sc=False),
        out_type=jax.ShapeDtypeStruct((n_pad_rows, 16), f32),
        scratch_types=[
            pltpu.VMEM((8, CHUNK), i32),
            pltpu.VMEM((CHUNK,), i32),
            pltpu.VMEM((CHUNK, 16), f32),
            pltpu.VMEM((DR, 16), f32),
            pltpu.VMEM_SHARED((n_pad_rows, 16), f32),
            pltpu.SemaphoreType.DMA,
        ],
    )
    def deg_kernel(packed_h, recip_h, hd_v, dump_v, buf_v, db_v, acc_sh, sem_s):
        c = lax.axis_index("c")
        s = lax.axis_index("s")

        @pl.when(c == 0)
        def _work():
            _fill(db_v, DR, 0.0)
            zbase = s * rows_pt
            for j in range(nd):
                pltpu.sync_copy(db_v, acc_sh.at[pl.ds(zbase + j * DR, DR)])
            plsc.subcore_barrier()

            _fill(buf_v, CHUNK, 1.0)
            def di(i, _):
                dump_v[pl.ds(i * 16, 16)] = jnp.full((16,), n_pad_rows - CHUNK, i32)
                return 0
            lax.fori_loop(0, CHUNK // 16, di, 0)
            def _wait_s():
                pltpu.make_async_copy(recip_h.at[pl.ds(0, CHUNK)], buf_v, sem_s).wait()
            pltpu.async_copy(buf_v, acc_sh.at[dump_v], sem_s, add=True)
            pltpu.async_copy(buf_v, acc_sh.at[dump_v], sem_s, add=True)

            bbase = s * n_blocks
            def body(d, _):
                _wait_s()
                _wait_s()
                pltpu.sync_copy(packed_h.at[bbase + d].at[pl.ds(16, 8)], hd_v)
                for k in range(8):
                    if k >= 2:
                        _wait_s()
                    pltpu.async_copy(buf_v, acc_sh.at[hd_v.at[k]], sem_s, add=True)
                return 0
            lax.fori_loop(0, n_blocks, body, 0)
            _wait_s()
            _wait_s()
            plsc.subcore_barrier()

            dbase = s * rows_pt
            def dbody(j, _):
                r = dbase + j * DR
                pltpu.sync_copy(acc_sh.at[pl.ds(r, DR)], db_v)
                def rb(i, _):
                    db_v[i, :] = 1.0 / jnp.maximum(db_v[i, :], 1.0)
                    return 0
                lax.fori_loop(0, DR, rb, 0)
                pltpu.sync_copy(db_v, recip_h.at[pl.ds(r, DR)])
                return 0
            lax.fori_loop(0, nd, dbody, 0)

    return deg_kernel


def _make_hop_kernel(n_pad_rows, n_blocks, emit_out):
    rows_pt = n_pad_rows // NSUB
    nd = rows_pt // DR

    if emit_out:
        out_types = [jax.ShapeDtypeStruct((2, n_pad_rows, H), f32),
                     jax.ShapeDtypeStruct((2, n_pad_rows, H), f32)]
    else:
        out_types = jax.ShapeDtypeStruct((2, n_pad_rows, H), f32)

    @functools.partial(
        pl.kernel, mesh=_mesh(),
        compiler_params=pltpu.CompilerParams(use_tc_tiling_on_sc=False),
        out_type=out_types,
        scratch_types=[
            pltpu.VMEM((24, CHUNK), i32),
            pltpu.VMEM((CHUNK,), i32),
            pltpu.VMEM((2, CHUNK, H), f32),
            pltpu.VMEM((2, CHUNK, H), f32),
            pltpu.VMEM((CHUNK, H), f32),
            pltpu.VMEM((DR, 16), f32),
            pltpu.VMEM_SHARED((n_pad_rows, H), f32),
            pltpu.SemaphoreType.DMA,
            pltpu.SemaphoreType.DMA,
        ],
    )
    def hop(*refs):
        if emit_out:
            (agg_h, rel_h, packed_h, recip_h, ego_h, agg1_h,
             out_agg, out_o3,
             idx_blk, dump_v, a_v, r_v, x_v, rc_v,
             acc_sh, sem_g, sem_s) = refs
        else:
            (agg_h, rel_h, packed_h, recip_h,
             out_agg,
             idx_blk, dump_v, a_v, r_v, x_v, rc_v,
             acc_sh, sem_g, sem_s) = refs
        c = lax.axis_index("c")
        s = lax.axis_index("s")

        _fill(x_v, CHUNK, 0.0)
        zbase = s * rows_pt
        for j in range(nd):
            pltpu.sync_copy(x_v, acc_sh.at[pl.ds(zbase + j * DR, DR)])
        plsc.subcore_barrier()

        def wait_g(p):
            pltpu.make_async_copy(agg_h.at[c].at[pl.ds(0, CHUNK)],
                                  a_v.at[p], sem_g).wait()
            pltpu.make_async_copy(agg_h.at[c].at[pl.ds(0, CHUNK)],
                                  r_v.at[p], sem_g).wait()

        def wait_s():
            pltpu.make_async_copy(agg_h.at[c].at[pl.ds(0, CHUNK)],
                                  a_v.at[0], sem_s).wait()

        def issue_g(k, p):
            pltpu.async_copy(agg_h.at[c].at[idx_blk.at[k]], a_v.at[p], sem_g)
            pltpu.async_copy(rel_h.at[c].at[idx_blk.at[8 + k]], r_v.at[p], sem_g)

        def di(i, _):
            dump_v[pl.ds(i * 16, 16)] = jnp.full((16,), n_pad_rows - CHUNK, i32)
            return 0
        lax.fori_loop(0, CHUNK // 16, di, 0)
        pltpu.async_copy(x_v, acc_sh.at[dump_v], sem_s, add=True)

        bbase = s * n_blocks
        def body(d, _):
            wait_s()
            pltpu.sync_copy(packed_h.at[bbase + d], idx_blk)
            issue_g(0, 0)
            for k in range(8):
                p = k & 1
                wait_g(p)
                if k:
                    wait_s()
                if k < 7:
                    issue_g(k + 1, 1 - p)
                def mul(i, _):
                    a_v[p, i, pl.ds(0, 16)] = (a_v[p, i, pl.ds(0, 16)]
                                               * r_v[p, i, pl.ds(0, 16)])
                    a_v[p, i, pl.ds(16, 16)] = (a_v[p, i, pl.ds(16, 16)]
                                                * r_v[p, i, pl.ds(16, 16)])
                    return 0
                lax.fori_loop(0, CHUNK, mul, 0)
                pltpu.async_copy(a_v.at[p], acc_sh.at[idx_blk.at[16 + k]],
                                 sem_s, add=True)
            return 0
        lax.fori_loop(0, n_blocks, body, 0)
        wait_s()
        plsc.subcore_barrier()

        dbase = s * rows_pt
        third = jnp.float32(1.0 / 3.0)
        def dbody(j, _):
            r = dbase + j * DR
            pltpu.sync_copy(acc_sh.at[pl.ds(r, DR)], a_v.at[0])
            pltpu.sync_copy(recip_h.at[pl.ds(r, DR)], rc_v)
            def rb(i, _):
                rr = rc_v[i, :]
                a_v[0, i, pl.ds(0, 16)] = a_v[0, i, pl.ds(0, 16)] * rr
                a_v[0, i, pl.ds(16, 16)] = a_v[0, i, pl.ds(16, 16)] * rr
                return 0
            lax.fori_loop(0, DR, rb, 0)
            pltpu.sync_copy(a_v.at[0], out_agg.at[c].at[pl.ds(r, DR)])
            if emit_out:
                pltpu.sync_copy(ego_h.at[c].at[pl.ds(r, DR)], r_v.at[0])
                pltpu.sync_copy(agg1_h.at[c].at[pl.ds(r, DR)], x_v)
                def ob(i, _):
                    lo = (a_v[0, i, pl.ds(0, 16)] + r_v[0, i, pl.ds(0, 16)]
                          + x_v[i, pl.ds(0, 16)]) * third
                    hi = (a_v[0, i, pl.ds(16, 16)] + r_v[0, i, pl.ds(16, 16)]
                          + x_v[i, pl.ds(16, 16)]) * third
                    r_v[0, i, pl.ds(0, 16)] = lo
                    r_v[0, i, pl.ds(16, 16)] = hi
                    return 0
                lax.fori_loop(0, DR, ob, 0)
                pltpu.sync_copy(r_v.at[0], out_o3.at[c].at[pl.ds(r, DR)])
            return 0
        lax.fori_loop(0, nd, dbody, 0)

    return hop


def _make_gather_kernel():
    per_sub = B // NSUB

    @functools.partial(
        pl.kernel, mesh=_mesh(),
        compiler_params=pltpu.CompilerParams(use_tc_tiling_on_sc=False),
        out_type=[jax.ShapeDtypeStruct((2, B, H), f32) for _ in range(6)],
        scratch_types=[
            pltpu.VMEM((CHUNK,), i32),
            pltpu.VMEM((CHUNK, H), f32),
            pltpu.SemaphoreType.DMA,
        ],
    )
    def gk(o3i, o3t, fui, fut, ul_h, uid_h, iid_h,
           ai, bi, at, bt, ii, it, idx_v, row_v, sem):
        c = lax.axis_index("c")
        s = lax.axis_index("s")
        base = s * per_sub
        for src, idxsrc, dst in ((fui, ul_h, ai), (o3i, uid_h, bi),
                                 (fut, ul_h, at), (o3t, uid_h, bt),
                                 (o3i, iid_h, ii), (o3t, iid_h, it)):
            for k in range(per_sub // CHUNK):
                b = base + k * CHUNK
                pltpu.sync_copy(idxsrc.at[pl.ds(b, CHUNK)], idx_v)
                pltpu.async_copy(src.at[c].at[idx_v], row_v, sem).wait()
                pltpu.sync_copy(row_v, dst.at[c].at[pl.ds(b, CHUNK)])

    return gk


def _mlp(x, w1t, b1, w2t, b2, block):
    n, k = x.shape
    h4 = w1t.shape[1]

    def body(x_ref, w1_ref, b1_ref, w2_ref, b2_ref, o_ref):
        h = jnp.dot(x_ref[...], w1_ref[...], preferred_element_type=f32) + b1_ref[...]
        h = jnp.where(h > 0, h, 0.01 * h)
        o_ref[...] = jnp.dot(h, w2_ref[...], preferred_element_type=f32) + b2_ref[...]

    return pl.pallas_call(
        body,
        grid=(n // block,),
        in_specs=[
            pl.BlockSpec((block, k), lambda i: (i, 0)),
            pl.BlockSpec((k, h4), lambda i: (0, 0)),
            pl.BlockSpec((1, h4), lambda i: (0, 0)),
            pl.BlockSpec((h4, D), lambda i: (0, 0)),
            pl.BlockSpec((1, D), lambda i: (0, 0)),
        ],
        out_specs=pl.BlockSpec((block, D), lambda i: (i, 0)),
        out_shape=jax.ShapeDtypeStruct((n, D), f32),
    )(x, w1t, b1, w2t, b2)


def _final(a_i, b_i, a_t, b_t, it_i, it_t, g1wt, g1b, g2wt, g2b, g3wt, g3b, g4wt, g4b):
    def sig(x):
        return 1.0 / (1.0 + jnp.exp(-x))

    def body(ai, bi, at, bt, ii, it, g1, c1, g2, c2, g3, c3, g4, c4, o_ref):
        av, bv = ai[...], bi[...]
        gate = sig(jnp.dot(av, g1[...], preferred_element_type=f32) + c1[...]
                   + jnp.dot(bv, g2[...], preferred_element_type=f32) + c2[...])
        uf_i = gate * av + (1.0 - gate) * bv
        av, bv = at[...], bt[...]
        gate = sig(jnp.dot(av, g3[...], preferred_element_type=f32) + c3[...]
                   + jnp.dot(bv, g4[...], preferred_element_type=f32) + c4[...])
        uf_t = gate * av + (1.0 - gate) * bv
        s = jnp.sum(uf_i * ii[...] + uf_t * it[...], axis=1)
        o_ref[...] = sig(s)

    return pl.pallas_call(
        body,
        out_shape=jax.ShapeDtypeStruct((B,), f32),
    )(a_i, b_i, a_t, b_t, it_i, it_t, g1wt, g1b, g2wt, g2b, g3wt, g3b, g4wt, g4b)


def _pack_edges(head, tail, etype, dump_row):
    group = NSUB * 8 * CHUNK
    e = head.shape[0]
    e_pad = -(-e // group) * group
    pad = e_pad - e
    head = jnp.concatenate([head, jnp.full((pad,), dump_row, i32)])
    tail = jnp.concatenate([tail, jnp.zeros((pad,), i32)])
    etype = jnp.concatenate([etype, jnp.zeros((pad,), i32)])
    packed = jnp.stack([tail.reshape(-1, 8, CHUNK), etype.reshape(-1, 8, CHUNK),
                        head.reshape(-1, 8, CHUNK)], axis=1).reshape(-1, 24, CHUNK)
    return packed, e_pad // group


def _halves(x):
    n = x.shape[0]
    return x.reshape(n, 2, H).transpose(1, 0, 2)


def _unhalves(x3):
    return x3.transpose(1, 0, 2).reshape(x3.shape[1], D)


def kernel(user_ids, item_ids, edge_index, edge_type, ukg_edge_index, ukg_edge_type,
           image_features, text_features,
           W_img1, b_img1, W_img2, b_img2, W_txt1, b_txt1, W_txt2, b_txt2,
           other_emb_image, other_emb_text, rel_emb_image, rel_emb_text,
           ukg_rel_emb_image, ukg_rel_emb_text,
           g1W, g1b, g2W, g2b, g3W, g3b, g4W, g4b):
    packed, n_blocks = _pack_edges(
        jnp.asarray(edge_index[0], i32), jnp.asarray(edge_index[1], i32),
        jnp.asarray(edge_type, i32), N_NODES)
    upacked, un_blocks = _pack_edges(
        jnp.asarray(ukg_edge_index[0], i32), jnp.asarray(ukg_edge_index[1], i32),
        jnp.asarray(ukg_edge_type, i32), N_USERS)

    n_pad = 51200
    u_pad = 10240

    img = _mlp(image_features, W_img1.T, b_img1[None, :], W_img2.T, b_img2[None, :], 2000)
    txt = _mlp(text_features, W_txt1.T, b_txt1[None, :], W_txt2.T, b_txt2[None, :], 2000)

    zpad = jnp.zeros((2, n_pad - N_NODES, H), f32)
    ego_i3 = jnp.concatenate([_halves(img), _halves(other_emb_image), zpad], axis=1)
    ego_t3 = jnp.concatenate([_halves(txt), _halves(other_emb_text), zpad], axis=1)
    rel_i3 = _halves(rel_emb_image)
    rel_t3 = _halves(rel_emb_text)
    urel_i3 = _halves(ukg_rel_emb_image)
    urel_t3 = _halves(ukg_rel_emb_text)

    recip_kg = _make_deg_kernel(n_pad, n_blocks)(packed)
    recip_ukg = _make_deg_kernel(u_pad, un_blocks)(upacked)

    hop1 = _make_hop_kernel(n_pad, n_blocks, False)
    hop2 = _make_hop_kernel(n_pad, n_blocks, True)
    agg1_i = hop1(ego_i3, rel_i3, packed, recip_kg)
    _, out_i3 = hop2(agg1_i, rel_i3, packed, recip_kg, ego_i3, agg1_i)
    agg1_t = hop1(ego_t3, rel_t3, packed, recip_kg)
    _, out_t3 = hop2(agg1_t, rel_t3, packed, recip_kg, ego_t3, agg1_t)

    uzpad = jnp.zeros((2, u_pad - N_USERS, H), f32)
    u_ego_i3 = jnp.concatenate(
        [out_i3[:, N_ENTITIES:N_NODES, :], uzpad], axis=1)
    u_ego_t3 = jnp.concatenate(
        [out_t3[:, N_ENTITIES:N_NODES, :], uzpad], axis=1)
    uhop1 = _make_hop_kernel(u_pad, un_blocks, False)
    uhop2 = _make_hop_kernel(u_pad, un_blocks, True)
    uagg1_i = uhop1(u_ego_i3, urel_i3, upacked, recip_ukg)
    _, fu_i3 = uhop2(uagg1_i, urel_i3, upacked, recip_ukg, u_ego_i3, uagg1_i)
    uagg1_t = uhop1(u_ego_t3, urel_t3, upacked, recip_ukg)
    _, fu_t3 = uhop2(uagg1_t, urel_t3, upacked, recip_ukg, u_ego_t3, uagg1_t)

    uid = jnp.asarray(user_ids, i32)
    ul = uid - N_ENTITIES
    iid = jnp.asarray(item_ids, i32)
    ai3, bi3, at3, bt3, ii3, it3 = _make_gather_kernel()(
        out_i3, out_t3, fu_i3, fu_t3, ul, uid, iid)

    return _final(_unhalves(ai3), _unhalves(bi3), _unhalves(at3), _unhalves(bt3),
                  _unhalves(ii3), _unhalves(it3),
                  g1W.T, g1b[None, :], g2W.T, g2b[None, :],
                  g3W.T, g3b[None, :], g4W.T, g4b[None, :])


# speedup vs baseline: 1.0157x; 1.0157x over previous
import functools

import jax
import jax.numpy as jnp
from jax import lax
from jax.experimental import pallas as pl
from jax.experimental.pallas import tpu as pltpu
from jax.experimental.pallas import tpu_sc as plsc

N_NODES = 50000
N_USERS = 10000
N_ENTITIES = 40000
D = 64
H = 32
CHUNK = 128
NSUB = 16
B = 4096
DR = 128

f32 = jnp.float32
i32 = jnp.int32


def _mesh():
    return plsc.VectorSubcoreMesh(core_axis_name="c", subcore_axis_name="s")


def _fill(ref, n_rows, value):
    k = ref.shape[1] // 16
    def body(i, _):
        for j in range(k):
            ref[i, pl.ds(16 * j, 16)] = jnp.full((16,), value, f32)
        return 0
    lax.fori_loop(0, n_rows, body, 0)


def _make_deg_kernel(n_pad_rows, n_blocks):
    rows_pt = n_pad_rows // NSUB
    nd = rows_pt // DR

    @functools.partial(
        pl.kernel, mesh=_mesh(),
        compiler_params=pltpu.CompilerParams(use_tc_tiling_on_sc=False),
        out_type=jax.ShapeDtypeStruct((n_pad_rows, 16), f32),
        scratch_types=[
            pltpu.VMEM((8, CHUNK), i32),
            pltpu.VMEM((CHUNK,), i32),
            pltpu.VMEM((CHUNK, 16), f32),
            pltpu.VMEM((DR, 16), f32),
            pltpu.VMEM_SHARED((n_pad_rows, 16), f32),
            pltpu.SemaphoreType.DMA,
        ],
    )
    def deg_kernel(packed_h, recip_h, hd_v, dump_v, buf_v, db_v, acc_sh, sem_s):
        c = lax.axis_index("c")
        s = lax.axis_index("s")

        @pl.when(c == 0)
        def _work():
            _fill(db_v, DR, 0.0)
            zbase = s * rows_pt
            for j in range(nd):
                pltpu.sync_copy(db_v, acc_sh.at[pl.ds(zbase + j * DR, DR)])
            plsc.subcore_barrier()

            _fill(buf_v, CHUNK, 1.0)
            def di(i, _):
                dump_v[pl.ds(i * 16, 16)] = jnp.full((16,), n_pad_rows - CHUNK, i32)
                return 0
            lax.fori_loop(0, CHUNK // 16, di, 0)
            def _wait_s():
                pltpu.make_async_copy(recip_h.at[pl.ds(0, CHUNK)], buf_v, sem_s).wait()
            pltpu.async_copy(buf_v, acc_sh.at[dump_v], sem_s, add=True)
            pltpu.async_copy(buf_v, acc_sh.at[dump_v], sem_s, add=True)

            bbase = s * n_blocks
            def body(d, _):
                _wait_s()
                _wait_s()
                pltpu.sync_copy(packed_h.at[bbase + d].at[pl.ds(16, 8)], hd_v)
                for k in range(8):
                    if k >= 2:
                        _wait_s()
                    pltpu.async_copy(buf_v, acc_sh.at[hd_v.at[k]], sem_s, add=True)
                return 0
            lax.fori_loop(0, n_blocks, body, 0)
            _wait_s()
            _wait_s()
            plsc.subcore_barrier()

            dbase = s * rows_pt
            def dbody(j, _):
                r = dbase + j * DR
                pltpu.sync_copy(acc_sh.at[pl.ds(r, DR)], db_v)
                def rb(i, _):
                    db_v[i, :] = 1.0 / jnp.maximum(db_v[i, :], 1.0)
                    return 0
                lax.fori_loop(0, DR, rb, 0)
                pltpu.sync_copy(db_v, recip_h.at[pl.ds(r, DR)])
                return 0
            lax.fori_loop(0, nd, dbody, 0)

    return deg_kernel


def _make_hop_kernel(n_pad_rows, n_blocks, emit_out):
    rows_pt = n_pad_rows // NSUB
    nd = rows_pt // DR

    if emit_out:
        out_types = [jax.ShapeDtypeStruct((2, n_pad_rows, H), f32),
                     jax.ShapeDtypeStruct((2, n_pad_rows, H), f32)]
    else:
        out_types = jax.ShapeDtypeStruct((2, n_pad_rows, H), f32)

    @functools.partial(
        pl.kernel, mesh=_mesh(),
        compiler_params=pltpu.CompilerParams(use_tc_tiling_on_sc=False),
        out_type=out_types,
        scratch_types=[
            pltpu.VMEM((24, CHUNK), i32),
            pltpu.VMEM((2, CHUNK, H), f32),
            pltpu.VMEM((2, CHUNK, H), f32),
            pltpu.VMEM((CHUNK, H), f32),
            pltpu.VMEM((DR, 16), f32),
            pltpu.VMEM_SHARED((n_pad_rows, H), f32),
            pltpu.SemaphoreType.DMA,
            pltpu.SemaphoreType.DMA,
        ],
    )
    def hop(*refs):
        if emit_out:
            (agg_h, rel_h, packed_h, recip_h, ego_h, agg1_h,
             out_agg, out_o3,
             idx_blk, a_v, r_v, x_v, rc_v,
             acc_sh, sem_g, sem_s) = refs
        else:
            (agg_h, rel_h, packed_h, recip_h,
             out_agg,
             idx_blk, a_v, r_v, x_v, rc_v,
             acc_sh, sem_g, sem_s) = refs
        c = lax.axis_index("c")
        s = lax.axis_index("s")

        _fill(x_v, CHUNK, 0.0)
        zbase = s * rows_pt
        for j in range(nd):
            pltpu.sync_copy(x_v, acc_sh.at[pl.ds(zbase + j * DR, DR)])
        plsc.subcore_barrier()

        bbase = s * n_blocks
        def body(d, _):
            pltpu.sync_copy(packed_h.at[bbase + d], idx_blk)
            for k in range(8):
                cp1 = pltpu.async_copy(agg_h.at[c].at[idx_blk.at[k]],
                                       a_v.at[0], sem_g)
                cp2 = pltpu.async_copy(rel_h.at[c].at[idx_blk.at[8 + k]],
                                       r_v.at[0], sem_s)
                cp1.wait()
                cp2.wait()
                def mul(i, _):
                    a_v[0, i, pl.ds(0, 16)] = (a_v[0, i, pl.ds(0, 16)]
                                               * r_v[0, i, pl.ds(0, 16)])
                    a_v[0, i, pl.ds(16, 16)] = (a_v[0, i, pl.ds(16, 16)]
                                                * r_v[0, i, pl.ds(16, 16)])
                    return 0
                lax.fori_loop(0, CHUNK, mul, 0)
                pltpu.sync_copy(a_v.at[0], acc_sh.at[idx_blk.at[16 + k]], add=True)
            return 0
        lax.fori_loop(0, n_blocks, body, 0)
        plsc.subcore_barrier()

        dbase = s * rows_pt
        third = jnp.float32(1.0 / 3.0)
        def dbody(j, _):
            r = dbase + j * DR
            pltpu.sync_copy(acc_sh.at[pl.ds(r, DR)], a_v.at[0])
            pltpu.sync_copy(recip_h.at[pl.ds(r, DR)], rc_v)
            def rb(i, _):
                rr = rc_v[i, :]
                a_v[0, i, pl.ds(0, 16)] = a_v[0, i, pl.ds(0, 16)] * rr
                a_v[0, i, pl.ds(16, 16)] = a_v[0, i, pl.ds(16, 16)] * rr
                return 0
            lax.fori_loop(0, DR, rb, 0)
            pltpu.sync_copy(a_v.at[0], out_agg.at[c].at[pl.ds(r, DR)])
            if emit_out:
                pltpu.sync_copy(ego_h.at[c].at[pl.ds(r, DR)], r_v.at[0])
                pltpu.sync_copy(agg1_h.at[c].at[pl.ds(r, DR)], x_v)
                def ob(i, _):
                    lo = (a_v[0, i, pl.ds(0, 16)] + r_v[0, i, pl.ds(0, 16)]
                          + x_v[i, pl.ds(0, 16)]) * third
                    hi = (a_v[0, i, pl.ds(16, 16)] + r_v[0, i, pl.ds(16, 16)]
                          + x_v[i, pl.ds(16, 16)]) * third
                    r_v[0, i, pl.ds(0, 16)] = lo
                    r_v[0, i, pl.ds(16, 16)] = hi
                    return 0
                lax.fori_loop(0, DR, ob, 0)
                pltpu.sync_copy(r_v.at[0], out_o3.at[c].at[pl.ds(r, DR)])
            return 0
        lax.fori_loop(0, nd, dbody, 0)

    return hop


def _make_gather_kernel():
    per_sub = B // NSUB

    @functools.partial(
        pl.kernel, mesh=_mesh(),
        compiler_params=pltpu.CompilerParams(use_tc_tiling_on_sc=False),
        out_type=[jax.ShapeDtypeStruct((2, B, H), f32) for _ in range(6)],
        scratch_types=[
            pltpu.VMEM((CHUNK,), i32),
            pltpu.VMEM((CHUNK, H), f32),
            pltpu.SemaphoreType.DMA,
        ],
    )
    def gk(o3i, o3t, fui, fut, ul_h, uid_h, iid_h,
           ai, bi, at, bt, ii, it, idx_v, row_v, sem):
        c = lax.axis_index("c")
        s = lax.axis_index("s")
        base = s * per_sub
        for src, idxsrc, dst in ((fui, ul_h, ai), (o3i, uid_h, bi),
                                 (fut, ul_h, at), (o3t, uid_h, bt),
                                 (o3i, iid_h, ii), (o3t, iid_h, it)):
            for k in range(per_sub // CHUNK):
                b = base + k * CHUNK
                pltpu.sync_copy(idxsrc.at[pl.ds(b, CHUNK)], idx_v)
                pltpu.async_copy(src.at[c].at[idx_v], row_v, sem).wait()
                pltpu.sync_copy(row_v, dst.at[c].at[pl.ds(b, CHUNK)])

    return gk


def _mlp(x, w1t, b1, w2t, b2, block):
    n, k = x.shape
    h4 = w1t.shape[1]

    def body(x_ref, w1_ref, b1_ref, w2_ref, b2_ref, o_ref):
        h = jnp.dot(x_ref[...], w1_ref[...], preferred_element_type=f32) + b1_ref[...]
        h = jnp.where(h > 0, h, 0.01 * h)
        o_ref[...] = jnp.dot(h, w2_ref[...], preferred_element_type=f32) + b2_ref[...]

    return pl.pallas_call(
        body,
        grid=(n // block,),
        in_specs=[
            pl.BlockSpec((block, k), lambda i: (i, 0)),
            pl.BlockSpec((k, h4), lambda i: (0, 0)),
            pl.BlockSpec((1, h4), lambda i: (0, 0)),
            pl.BlockSpec((h4, D), lambda i: (0, 0)),
            pl.BlockSpec((1, D), lambda i: (0, 0)),
        ],
        out_specs=pl.BlockSpec((block, D), lambda i: (i, 0)),
        out_shape=jax.ShapeDtypeStruct((n, D), f32),
    )(x, w1t, b1, w2t, b2)


def _final(a_i, b_i, a_t, b_t, it_i, it_t, g1wt, g1b, g2wt, g2b, g3wt, g3b, g4wt, g4b):
    def sig(x):
        return 1.0 / (1.0 + jnp.exp(-x))

    def body(ai, bi, at, bt, ii, it, g1, c1, g2, c2, g3, c3, g4, c4, o_ref):
        av, bv = ai[...], bi[...]
        gate = sig(jnp.dot(av, g1[...], preferred_element_type=f32) + c1[...]
                   + jnp.dot(bv, g2[...], preferred_element_type=f32) + c2[...])
        uf_i = gate * av + (1.0 - gate) * bv
        av, bv = at[...], bt[...]
        gate = sig(jnp.dot(av, g3[...], preferred_element_type=f32) + c3[...]
                   + jnp.dot(bv, g4[...], preferred_element_type=f32) + c4[...])
        uf_t = gate * av + (1.0 - gate) * bv
        s = jnp.sum(uf_i * ii[...] + uf_t * it[...], axis=1)
        o_ref[...] = sig(s)

    return pl.pallas_call(
        body,
        out_shape=jax.ShapeDtypeStruct((B,), f32),
    )(a_i, b_i, a_t, b_t, it_i, it_t, g1wt, g1b, g2wt, g2b, g3wt, g3b, g4wt, g4b)


def _pack_edges(head, tail, etype, dump_row):
    group = NSUB * 8 * CHUNK
    e = head.shape[0]
    e_pad = -(-e // group) * group
    pad = e_pad - e
    head = jnp.concatenate([head, jnp.full((pad,), dump_row, i32)])
    tail = jnp.concatenate([tail, jnp.zeros((pad,), i32)])
    etype = jnp.concatenate([etype, jnp.zeros((pad,), i32)])
    packed = jnp.stack([tail.reshape(-1, 8, CHUNK), etype.reshape(-1, 8, CHUNK),
                        head.reshape(-1, 8, CHUNK)], axis=1).reshape(-1, 24, CHUNK)
    return packed, e_pad // group


def _halves(x):
    n = x.shape[0]
    return x.reshape(n, 2, H).transpose(1, 0, 2)


def _unhalves(x3):
    return x3.transpose(1, 0, 2).reshape(x3.shape[1], D)


def kernel(user_ids, item_ids, edge_index, edge_type, ukg_edge_index, ukg_edge_type,
           image_features, text_features,
           W_img1, b_img1, W_img2, b_img2, W_txt1, b_txt1, W_txt2, b_txt2,
           other_emb_image, other_emb_text, rel_emb_image, rel_emb_text,
           ukg_rel_emb_image, ukg_rel_emb_text,
           g1W, g1b, g2W, g2b, g3W, g3b, g4W, g4b):
    packed, n_blocks = _pack_edges(
        jnp.asarray(edge_index[0], i32), jnp.asarray(edge_index[1], i32),
        jnp.asarray(edge_type, i32), N_NODES)
    upacked, un_blocks = _pack_edges(
        jnp.asarray(ukg_edge_index[0], i32), jnp.asarray(ukg_edge_index[1], i32),
        jnp.asarray(ukg_edge_type, i32), N_USERS)

    n_pad = 51200
    u_pad = 10240

    img = _mlp(image_features, W_img1.T, b_img1[None, :], W_img2.T, b_img2[None, :], 2000)
    txt = _mlp(text_features, W_txt1.T, b_txt1[None, :], W_txt2.T, b_txt2[None, :], 2000)

    zpad = jnp.zeros((2, n_pad - N_NODES, H), f32)
    ego_i3 = jnp.concatenate([_halves(img), _halves(other_emb_image), zpad], axis=1)
    ego_t3 = jnp.concatenate([_halves(txt), _halves(other_emb_text), zpad], axis=1)
    rel_i3 = _halves(rel_emb_image)
    rel_t3 = _halves(rel_emb_text)
    urel_i3 = _halves(ukg_rel_emb_image)
    urel_t3 = _halves(ukg_rel_emb_text)

    recip_kg = _make_deg_kernel(n_pad, n_blocks)(packed)
    recip_ukg = _make_deg_kernel(u_pad, un_blocks)(upacked)

    hop1 = _make_hop_kernel(n_pad, n_blocks, False)
    hop2 = _make_hop_kernel(n_pad, n_blocks, True)
    agg1_i = hop1(ego_i3, rel_i3, packed, recip_kg)
    _, out_i3 = hop2(agg1_i, rel_i3, packed, recip_kg, ego_i3, agg1_i)
    agg1_t = hop1(ego_t3, rel_t3, packed, recip_kg)
    _, out_t3 = hop2(agg1_t, rel_t3, packed, recip_kg, ego_t3, agg1_t)

    uzpad = jnp.zeros((2, u_pad - N_USERS, H), f32)
    u_ego_i3 = jnp.concatenate(
        [out_i3[:, N_ENTITIES:N_NODES, :], uzpad], axis=1)
    u_ego_t3 = jnp.concatenate(
        [out_t3[:, N_ENTITIES:N_NODES, :], uzpad], axis=1)
    uhop1 = _make_hop_kernel(u_pad, un_blocks, False)
    uhop2 = _make_hop_kernel(u_pad, un_blocks, True)
    uagg1_i = uhop1(u_ego_i3, urel_i3, upacked, recip_ukg)
    _, fu_i3 = uhop2(uagg1_i, urel_i3, upacked, recip_ukg, u_ego_i3, uagg1_i)
    uagg1_t = uhop1(u_ego_t3, urel_t3, upacked, recip_ukg)
    _, fu_t3 = uhop2(uagg1_t, urel_t3, upacked, recip_ukg, u_ego_t3, uagg1_t)

    uid = jnp.asarray(user_ids, i32)
    ul = uid - N_ENTITIES
    iid = jnp.asarray(item_ids, i32)
    ai3, bi3, at3, bt3, ii3, it3 = _make_gather_kernel()(
        out_i3, out_t3, fu_i3, fu_t3, ul, uid, iid)

    return _final(_unhalves(ai3), _unhalves(bi3), _unhalves(at3), _unhalves(bt3),
                  _unhalves(ii3), _unhalves(it3),
                  g1W.T, g1b[None, :], g2W.T, g2b[None, :],
                  g3W.T, g3b[None, :], g4W.T, g4b[None, :])


# speedup vs baseline: 1.0866x; 1.0699x over previous
import functools

import jax
import jax.numpy as jnp
from jax import lax
from jax.experimental import pallas as pl
from jax.experimental.pallas import tpu as pltpu
from jax.experimental.pallas import tpu_sc as plsc

N_NODES = 50000
N_USERS = 10000
N_ENTITIES = 40000
D = 64
H = 32
CHUNK = 128
NSUB = 16
B = 4096

f32 = jnp.float32
i32 = jnp.int32


def _mesh():
    return plsc.VectorSubcoreMesh(core_axis_name="c", subcore_axis_name="s")


def _fill(ref, n_rows, value):
    k = ref.shape[1] // 16
    def body(i, _):
        for j in range(k):
            ref[i, pl.ds(16 * j, 16)] = jnp.full((16,), value, f32)
        return 0
    lax.fori_loop(0, n_rows, body, 0)


def _make_deg_kernel(n_pad_rows, e_pad):
    n_chunks = e_pad // (NSUB * CHUNK)
    rows_pt = n_pad_rows // NSUB
    nd = rows_pt // CHUNK

    @functools.partial(
        pl.kernel, mesh=_mesh(),
        compiler_params=pltpu.CompilerParams(use_tc_tiling_on_sc=False),
        out_type=jax.ShapeDtypeStruct((n_pad_rows, 16), f32),
        scratch_types=[
            pltpu.VMEM((CHUNK,), i32),
            pltpu.VMEM((CHUNK, 16), f32),
            pltpu.VMEM_SHARED((n_pad_rows, 16), f32),
        ],
    )
    def deg_kernel(head_h, recip_h, idx_v, buf_v, acc_sh):
        c = lax.axis_index("c")
        s = lax.axis_index("s")

        @pl.when(c == 0)
        def _work():
            _fill(buf_v, CHUNK, 0.0)
            zbase = s * rows_pt
            for j in range(nd):
                pltpu.sync_copy(buf_v, acc_sh.at[pl.ds(zbase + j * CHUNK, CHUNK)])
            plsc.subcore_barrier()

            _fill(buf_v, CHUNK, 1.0)
            ebase = s * (n_chunks * CHUNK)
            def body(j, _):
                pltpu.sync_copy(head_h.at[pl.ds(ebase + j * CHUNK, CHUNK)], idx_v)
                pltpu.sync_copy(buf_v, acc_sh.at[idx_v], add=True)
                return 0
            lax.fori_loop(0, n_chunks, body, 0)
            plsc.subcore_barrier()

            dbase = s * rows_pt
            def dbody(j, _):
                r = dbase + j * CHUNK
                pltpu.sync_copy(acc_sh.at[pl.ds(r, CHUNK)], buf_v)
                def rb(i, _):
                    buf_v[i, :] = 1.0 / jnp.maximum(buf_v[i, :], 1.0)
                    return 0
                lax.fori_loop(0, CHUNK, rb, 0)
                pltpu.sync_copy(buf_v, recip_h.at[pl.ds(r, CHUNK)])
                return 0
            lax.fori_loop(0, nd, dbody, 0)

    return deg_kernel


def _make_hop_kernel(n_pad_rows, e_pad, emit_out):
    n_chunks = e_pad // (NSUB * CHUNK)
    rows_pt = n_pad_rows // NSUB
    nd = rows_pt // CHUNK

    if emit_out:
        out_types = [jax.ShapeDtypeStruct((2, n_pad_rows, H), f32),
                     jax.ShapeDtypeStruct((2, n_pad_rows, H), f32)]
    else:
        out_types = jax.ShapeDtypeStruct((2, n_pad_rows, H), f32)

    @functools.partial(
        pl.kernel, mesh=_mesh(),
        compiler_params=pltpu.CompilerParams(use_tc_tiling_on_sc=False),
        out_type=out_types,
        scratch_types=[
            pltpu.VMEM((CHUNK,), i32),
            pltpu.VMEM((CHUNK,), i32),
            pltpu.VMEM((CHUNK,), i32),
            pltpu.VMEM((CHUNK, H), f32),
            pltpu.VMEM((CHUNK, H), f32),
            pltpu.VMEM((CHUNK, H), f32),
            pltpu.VMEM((CHUNK, 16), f32),
            pltpu.VMEM_SHARED((n_pad_rows, H), f32),
            pltpu.SemaphoreType.DMA,
            pltpu.SemaphoreType.DMA,
        ],
    )
    def hop(*refs):
        if emit_out:
            (agg_h, rel_h, head_h, tail_h, et_h, recip_h, ego_h, agg1_h,
             out_agg, out_o3,
             tidx, hidx, eidx, a_v, r_v, x_v, rc_v, acc_sh, sem, sem2) = refs
        else:
            (agg_h, rel_h, head_h, tail_h, et_h, recip_h,
             out_agg,
             tidx, hidx, eidx, a_v, r_v, x_v, rc_v, acc_sh, sem, sem2) = refs
        c = lax.axis_index("c")
        s = lax.axis_index("s")

        _fill(x_v, CHUNK, 0.0)
        zbase = s * rows_pt
        for j in range(nd):
            pltpu.sync_copy(x_v, acc_sh.at[pl.ds(zbase + j * CHUNK, CHUNK)])
        plsc.subcore_barrier()

        ebase = s * (n_chunks * CHUNK)
        def body(j, _):
            b = ebase + j * CHUNK
            pltpu.sync_copy(tail_h.at[pl.ds(b, CHUNK)], tidx)
            pltpu.sync_copy(et_h.at[pl.ds(b, CHUNK)], eidx)
            pltpu.sync_copy(head_h.at[pl.ds(b, CHUNK)], hidx)
            cp1 = pltpu.async_copy(agg_h.at[c].at[tidx], a_v, sem)
            cp2 = pltpu.async_copy(rel_h.at[c].at[eidx], r_v, sem2)
            cp1.wait()
            cp2.wait()
            def mul(i, _):
                a_v[i, pl.ds(0, 16)] = a_v[i, pl.ds(0, 16)] * r_v[i, pl.ds(0, 16)]
                a_v[i, pl.ds(16, 16)] = a_v[i, pl.ds(16, 16)] * r_v[i, pl.ds(16, 16)]
                return 0
            lax.fori_loop(0, CHUNK, mul, 0)
            pltpu.sync_copy(a_v, acc_sh.at[hidx], add=True)
            return 0
        lax.fori_loop(0, n_chunks, body, 0)
        plsc.subcore_barrier()

        dbase = s * rows_pt
        third = jnp.float32(1.0 / 3.0)
        def dbody(j, _):
            r = dbase + j * CHUNK
            pltpu.sync_copy(acc_sh.at[pl.ds(r, CHUNK)], a_v)
            pltpu.sync_copy(recip_h.at[pl.ds(r, CHUNK)], rc_v)
            def rb(i, _):
                rr = rc_v[i, :]
                a_v[i, pl.ds(0, 16)] = a_v[i, pl.ds(0, 16)] * rr
                a_v[i, pl.ds(16, 16)] = a_v[i, pl.ds(16, 16)] * rr
                return 0
            lax.fori_loop(0, CHUNK, rb, 0)
            pltpu.sync_copy(a_v, out_agg.at[c].at[pl.ds(r, CHUNK)])
            if emit_out:
                pltpu.sync_copy(ego_h.at[c].at[pl.ds(r, CHUNK)], r_v)
                pltpu.sync_copy(agg1_h.at[c].at[pl.ds(r, CHUNK)], x_v)
                def ob(i, _):
                    lo = (a_v[i, pl.ds(0, 16)] + r_v[i, pl.ds(0, 16)]
                          + x_v[i, pl.ds(0, 16)]) * third
                    hi = (a_v[i, pl.ds(16, 16)] + r_v[i, pl.ds(16, 16)]
                          + x_v[i, pl.ds(16, 16)]) * third
                    r_v[i, pl.ds(0, 16)] = lo
                    r_v[i, pl.ds(16, 16)] = hi
                    return 0
                lax.fori_loop(0, CHUNK, ob, 0)
                pltpu.sync_copy(r_v, out_o3.at[c].at[pl.ds(r, CHUNK)])
            return 0
        lax.fori_loop(0, nd, dbody, 0)

    return hop


def _make_gather_kernel():
    per_sub = B // NSUB

    @functools.partial(
        pl.kernel, mesh=_mesh(),
        compiler_params=pltpu.CompilerParams(use_tc_tiling_on_sc=False),
        out_type=[jax.ShapeDtypeStruct((2, B, H), f32) for _ in range(6)],
        scratch_types=[
            pltpu.VMEM((CHUNK,), i32),
            pltpu.VMEM((CHUNK, H), f32),
            pltpu.SemaphoreType.DMA,
        ],
    )
    def gk(o3i, o3t, fui, fut, ul_h, uid_h, iid_h,
           ai, bi, at, bt, ii, it, idx_v, row_v, sem):
        c = lax.axis_index("c")
        s = lax.axis_index("s")
        base = s * per_sub
        for src, idxsrc, dst in ((fui, ul_h, ai), (o3i, uid_h, bi),
                                 (fut, ul_h, at), (o3t, uid_h, bt),
                                 (o3i, iid_h, ii), (o3t, iid_h, it)):
            for k in range(per_sub // CHUNK):
                b = base + k * CHUNK
                pltpu.sync_copy(idxsrc.at[pl.ds(b, CHUNK)], idx_v)
                pltpu.async_copy(src.at[c].at[idx_v], row_v, sem).wait()
                pltpu.sync_copy(row_v, dst.at[c].at[pl.ds(b, CHUNK)])

    return gk


def _mlp(x, w1t, b1, w2t, b2, block):
    n, k = x.shape
    h4 = w1t.shape[1]

    def body(x_ref, w1_ref, b1_ref, w2_ref, b2_ref, o_ref):
        h = jnp.dot(x_ref[...], w1_ref[...], preferred_element_type=f32) + b1_ref[...]
        h = jnp.where(h > 0, h, 0.01 * h)
        o_ref[...] = jnp.dot(h, w2_ref[...], preferred_element_type=f32) + b2_ref[...]

    return pl.pallas_call(
        body,
        grid=(n // block,),
        in_specs=[
            pl.BlockSpec((block, k), lambda i: (i, 0)),
            pl.BlockSpec((k, h4), lambda i: (0, 0)),
            pl.BlockSpec((1, h4), lambda i: (0, 0)),
            pl.BlockSpec((h4, D), lambda i: (0, 0)),
            pl.BlockSpec((1, D), lambda i: (0, 0)),
        ],
        out_specs=pl.BlockSpec((block, D), lambda i: (i, 0)),
        out_shape=jax.ShapeDtypeStruct((n, D), f32),
    )(x, w1t, b1, w2t, b2)


def _final(a_i, b_i, a_t, b_t, it_i, it_t, g1wt, g1b, g2wt, g2b, g3wt, g3b, g4wt, g4b):
    def sig(x):
        return 1.0 / (1.0 + jnp.exp(-x))

    def body(ai, bi, at, bt, ii, it, g1, c1, g2, c2, g3, c3, g4, c4, o_ref):
        av, bv = ai[...], bi[...]
        gate = sig(jnp.dot(av, g1[...], preferred_element_type=f32) + c1[...]
                   + jnp.dot(bv, g2[...], preferred_element_type=f32) + c2[...])
        uf_i = gate * av + (1.0 - gate) * bv
        av, bv = at[...], bt[...]
        gate = sig(jnp.dot(av, g3[...], preferred_element_type=f32) + c3[...]
                   + jnp.dot(bv, g4[...], preferred_element_type=f32) + c4[...])
        uf_t = gate * av + (1.0 - gate) * bv
        s = jnp.sum(uf_i * ii[...] + uf_t * it[...], axis=1)
        o_ref[...] = sig(s)

    return pl.pallas_call(
        body,
        out_shape=jax.ShapeDtypeStruct((B,), f32),
    )(a_i, b_i, a_t, b_t, it_i, it_t, g1wt, g1b, g2wt, g2b, g3wt, g3b, g4wt, g4b)


def _pad_edges(head, tail, etype, dump_row, group):
    e = head.shape[0]
    e_pad = -(-e // group) * group
    pad = e_pad - e
    head = jnp.concatenate([head, jnp.full((pad,), dump_row, i32)])
    tail = jnp.concatenate([tail, jnp.zeros((pad,), i32)])
    etype = jnp.concatenate([etype, jnp.zeros((pad,), i32)])
    return head, tail, etype, e_pad


def _halves(x):
    n = x.shape[0]
    return x.reshape(n, 2, H).transpose(1, 0, 2)


def _unhalves(x3):
    return x3.transpose(1, 0, 2).reshape(x3.shape[1], D)


def kernel(user_ids, item_ids, edge_index, edge_type, ukg_edge_index, ukg_edge_type,
           image_features, text_features,
           W_img1, b_img1, W_img2, b_img2, W_txt1, b_txt1, W_txt2, b_txt2,
           other_emb_image, other_emb_text, rel_emb_image, rel_emb_text,
           ukg_rel_emb_image, ukg_rel_emb_text,
           g1W, g1b, g2W, g2b, g3W, g3b, g4W, g4b):
    group = NSUB * CHUNK
    head = jnp.asarray(edge_index[0], i32)
    tail = jnp.asarray(edge_index[1], i32)
    et = jnp.asarray(edge_type, i32)
    uhead = jnp.asarray(ukg_edge_index[0], i32)
    utail = jnp.asarray(ukg_edge_index[1], i32)
    uet = jnp.asarray(ukg_edge_type, i32)

    head, tail, et, e_pad = _pad_edges(head, tail, et, N_NODES, group)
    uhead, utail, uet, ue_pad = _pad_edges(uhead, utail, uet, N_USERS, group)

    n_pad = 51200
    u_pad = 10240

    img = _mlp(image_features, W_img1.T, b_img1[None, :], W_img2.T, b_img2[None, :], 2000)
    txt = _mlp(text_features, W_txt1.T, b_txt1[None, :], W_txt2.T, b_txt2[None, :], 2000)

    zpad = jnp.zeros((2, n_pad - N_NODES, H), f32)
    ego_i3 = jnp.concatenate([_halves(img), _halves(other_emb_image), zpad], axis=1)
    ego_t3 = jnp.concatenate([_halves(txt), _halves(other_emb_text), zpad], axis=1)
    rel_i3 = _halves(rel_emb_image)
    rel_t3 = _halves(rel_emb_text)
    urel_i3 = _halves(ukg_rel_emb_image)
    urel_t3 = _halves(ukg_rel_emb_text)

    recip_kg = _make_deg_kernel(n_pad, e_pad)(head)
    recip_ukg = _make_deg_kernel(u_pad, ue_pad)(uhead)

    hop1 = _make_hop_kernel(n_pad, e_pad, False)
    hop2 = _make_hop_kernel(n_pad, e_pad, True)
    agg1_i = hop1(ego_i3, rel_i3, head, tail, et, recip_kg)
    _, out_i3 = hop2(agg1_i, rel_i3, head, tail, et, recip_kg, ego_i3, agg1_i)
    agg1_t = hop1(ego_t3, rel_t3, head, tail, et, recip_kg)
    _, out_t3 = hop2(agg1_t, rel_t3, head, tail, et, recip_kg, ego_t3, agg1_t)

    uzpad = jnp.zeros((2, u_pad - N_USERS, H), f32)
    u_ego_i3 = jnp.concatenate(
        [out_i3[:, N_ENTITIES:N_NODES, :], uzpad], axis=1)
    u_ego_t3 = jnp.concatenate(
        [out_t3[:, N_ENTITIES:N_NODES, :], uzpad], axis=1)
    uhop1 = _make_hop_kernel(u_pad, ue_pad, False)
    uhop2 = _make_hop_kernel(u_pad, ue_pad, True)
    uagg1_i = uhop1(u_ego_i3, urel_i3, uhead, utail, uet, recip_ukg)
    _, fu_i3 = uhop2(uagg1_i, urel_i3, uhead, utail, uet, recip_ukg, u_ego_i3, uagg1_i)
    uagg1_t = uhop1(u_ego_t3, urel_t3, uhead, utail, uet, recip_ukg)
    _, fu_t3 = uhop2(uagg1_t, urel_t3, uhead, utail, uet, recip_ukg, u_ego_t3, uagg1_t)

    uid = jnp.asarray(user_ids, i32)
    ul = uid - N_ENTITIES
    iid = jnp.asarray(item_ids, i32)
    ai3, bi3, at3, bt3, ii3, it3 = _make_gather_kernel()(
        out_i3, out_t3, fu_i3, fu_t3, ul, uid, iid)

    return _final(_unhalves(ai3), _unhalves(bi3), _unhalves(at3), _unhalves(bt3),
                  _unhalves(ii3), _unhalves(it3),
                  g1W.T, g1b[None, :], g2W.T, g2b[None, :],
                  g3W.T, g3b[None, :], g4W.T, g4b[None, :])


# speedup vs baseline: 1.1948x; 1.0995x over previous
import functools

import jax
import jax.numpy as jnp
from jax import lax
from jax.experimental import pallas as pl
from jax.experimental.pallas import tpu as pltpu
from jax.experimental.pallas import tpu_sc as plsc

N_NODES = 50000
N_USERS = 10000
N_ENTITIES = 40000
D = 64
H = 32
CHUNK = 128
NSUB = 16
B = 4096

f32 = jnp.float32
i32 = jnp.int32


def _mesh():
    return plsc.VectorSubcoreMesh(core_axis_name="c", subcore_axis_name="s")


def _fill(ref, n_rows, value):
    k = ref.shape[1] // 16
    def body(i, _):
        for j in range(k):
            ref[i, pl.ds(16 * j, 16)] = jnp.full((16,), value, f32)
        return 0
    lax.fori_loop(0, n_rows, body, 0)


def _make_deg_kernel(n_pad_rows, e_pad):
    n_chunks = e_pad // (NSUB * CHUNK)
    rows_pt = n_pad_rows // NSUB
    nd = rows_pt // CHUNK

    @functools.partial(
        pl.kernel, mesh=_mesh(),
        compiler_params=pltpu.CompilerParams(use_tc_tiling_on_sc=False),
        out_type=jax.ShapeDtypeStruct((n_pad_rows, 16), f32),
        scratch_types=[
            pltpu.VMEM((CHUNK,), i32),
            pltpu.VMEM((CHUNK, 16), f32),
            pltpu.VMEM_SHARED((n_pad_rows, 16), f32),
        ],
    )
    def deg_kernel(head_h, recip_h, idx_v, buf_v, acc_sh):
        c = lax.axis_index("c")
        s = lax.axis_index("s")

        @pl.when(c == 0)
        def _work():
            _fill(buf_v, CHUNK, 0.0)
            zbase = s * rows_pt
            for j in range(nd):
                pltpu.sync_copy(buf_v, acc_sh.at[pl.ds(zbase + j * CHUNK, CHUNK)])
            plsc.subcore_barrier()

            _fill(buf_v, CHUNK, 1.0)
            ebase = s * (n_chunks * CHUNK)
            def body(j, _):
                pltpu.sync_copy(head_h.at[pl.ds(ebase + j * CHUNK, CHUNK)], idx_v)
                pltpu.sync_copy(buf_v, acc_sh.at[idx_v], add=True)
                return 0
            lax.fori_loop(0, n_chunks, body, 0)
            plsc.subcore_barrier()

            dbase = s * rows_pt
            def dbody(j, _):
                r = dbase + j * CHUNK
                pltpu.sync_copy(acc_sh.at[pl.ds(r, CHUNK)], buf_v)
                def rb(i, _):
                    buf_v[i, :] = 1.0 / jnp.maximum(buf_v[i, :], 1.0)
                    return 0
                lax.fori_loop(0, CHUNK, rb, 0)
                pltpu.sync_copy(buf_v, recip_h.at[pl.ds(r, CHUNK)])
                return 0
            lax.fori_loop(0, nd, dbody, 0)

    return deg_kernel


def _make_hop_kernel(n_pad_rows, e_pad, emit_out):
    n_chunks = e_pad // (NSUB * CHUNK)
    rows_pt = n_pad_rows // NSUB
    nd = rows_pt // CHUNK

    if emit_out:
        out_types = [jax.ShapeDtypeStruct((2, n_pad_rows, H), f32),
                     jax.ShapeDtypeStruct((2, n_pad_rows, H), f32)]
    else:
        out_types = jax.ShapeDtypeStruct((2, n_pad_rows, H), f32)

    @functools.partial(
        pl.kernel, mesh=_mesh(),
        compiler_params=pltpu.CompilerParams(use_tc_tiling_on_sc=False),
        out_type=out_types,
        scratch_types=[
            pltpu.VMEM((CHUNK,), i32),
            pltpu.VMEM((CHUNK,), i32),
            pltpu.VMEM((CHUNK,), i32),
            pltpu.VMEM((CHUNK, H), f32),
            pltpu.VMEM((CHUNK, H), f32),
            pltpu.VMEM((CHUNK, H), f32),
            pltpu.VMEM((CHUNK, 16), f32),
            pltpu.VMEM_SHARED((n_pad_rows, H), f32),
            pltpu.SemaphoreType.DMA,
            pltpu.SemaphoreType.DMA,
            pltpu.SemaphoreType.DMA,
        ],
    )
    def hop(*refs):
        if emit_out:
            (agg_h, rel_h, head_h, tail_h, et_h, recip_h, ego_h, agg1_h,
             out_agg, out_o3,
             tidx, hidx, eidx, a_v, r_v, x_v, rc_v, acc_sh, sem, sem2, sem3) = refs
        else:
            (agg_h, rel_h, head_h, tail_h, et_h, recip_h,
             out_agg,
             tidx, hidx, eidx, a_v, r_v, x_v, rc_v, acc_sh, sem, sem2, sem3) = refs
        c = lax.axis_index("c")
        s = lax.axis_index("s")

        _fill(x_v, CHUNK, 0.0)
        zbase = s * rows_pt
        for j in range(nd):
            pltpu.sync_copy(x_v, acc_sh.at[pl.ds(zbase + j * CHUNK, CHUNK)])
        plsc.subcore_barrier()

        ebase = s * (n_chunks * CHUNK)
        def body(j, _):
            b = ebase + j * CHUNK
            ci1 = pltpu.async_copy(tail_h.at[pl.ds(b, CHUNK)], tidx, sem3)
            ci2 = pltpu.async_copy(et_h.at[pl.ds(b, CHUNK)], eidx, sem3)
            ci3 = pltpu.async_copy(head_h.at[pl.ds(b, CHUNK)], hidx, sem3)
            ci1.wait()
            ci2.wait()
            ci3.wait()
            cp1 = pltpu.async_copy(agg_h.at[c].at[tidx], a_v, sem)
            cp2 = pltpu.async_copy(rel_h.at[c].at[eidx], r_v, sem2)
            cp1.wait()
            cp2.wait()
            def mul(i, _):
                a_v[i, pl.ds(0, 16)] = a_v[i, pl.ds(0, 16)] * r_v[i, pl.ds(0, 16)]
                a_v[i, pl.ds(16, 16)] = a_v[i, pl.ds(16, 16)] * r_v[i, pl.ds(16, 16)]
                return 0
            lax.fori_loop(0, CHUNK, mul, 0)
            pltpu.sync_copy(a_v, acc_sh.at[hidx], add=True)
            return 0
        lax.fori_loop(0, n_chunks, body, 0)
        plsc.subcore_barrier()

        dbase = s * rows_pt
        third = jnp.float32(1.0 / 3.0)
        def dbody(j, _):
            r = dbase + j * CHUNK
            pltpu.sync_copy(acc_sh.at[pl.ds(r, CHUNK)], a_v)
            pltpu.sync_copy(recip_h.at[pl.ds(r, CHUNK)], rc_v)
            def rb(i, _):
                rr = rc_v[i, :]
                a_v[i, pl.ds(0, 16)] = a_v[i, pl.ds(0, 16)] * rr
                a_v[i, pl.ds(16, 16)] = a_v[i, pl.ds(16, 16)] * rr
                return 0
            lax.fori_loop(0, CHUNK, rb, 0)
            pltpu.sync_copy(a_v, out_agg.at[c].at[pl.ds(r, CHUNK)])
            if emit_out:
                pltpu.sync_copy(ego_h.at[c].at[pl.ds(r, CHUNK)], r_v)
                pltpu.sync_copy(agg1_h.at[c].at[pl.ds(r, CHUNK)], x_v)
                def ob(i, _):
                    lo = (a_v[i, pl.ds(0, 16)] + r_v[i, pl.ds(0, 16)]
                          + x_v[i, pl.ds(0, 16)]) * third
                    hi = (a_v[i, pl.ds(16, 16)] + r_v[i, pl.ds(16, 16)]
                          + x_v[i, pl.ds(16, 16)]) * third
                    r_v[i, pl.ds(0, 16)] = lo
                    r_v[i, pl.ds(16, 16)] = hi
                    return 0
                lax.fori_loop(0, CHUNK, ob, 0)
                pltpu.sync_copy(r_v, out_o3.at[c].at[pl.ds(r, CHUNK)])
            return 0
        lax.fori_loop(0, nd, dbody, 0)

    return hop


def _make_gather_kernel():
    per_sub = B // NSUB

    @functools.partial(
        pl.kernel, mesh=_mesh(),
        compiler_params=pltpu.CompilerParams(use_tc_tiling_on_sc=False),
        out_type=[jax.ShapeDtypeStruct((2, B, H), f32) for _ in range(6)],
        scratch_types=[
            pltpu.VMEM((CHUNK,), i32),
            pltpu.VMEM((CHUNK, H), f32),
            pltpu.SemaphoreType.DMA,
        ],
    )
    def gk(o3i, o3t, fui, fut, ul_h, uid_h, iid_h,
           ai, bi, at, bt, ii, it, idx_v, row_v, sem):
        c = lax.axis_index("c")
        s = lax.axis_index("s")
        base = s * per_sub
        for src, idxsrc, dst in ((fui, ul_h, ai), (o3i, uid_h, bi),
                                 (fut, ul_h, at), (o3t, uid_h, bt),
                                 (o3i, iid_h, ii), (o3t, iid_h, it)):
            for k in range(per_sub // CHUNK):
                b = base + k * CHUNK
                pltpu.sync_copy(idxsrc.at[pl.ds(b, CHUNK)], idx_v)
                pltpu.async_copy(src.at[c].at[idx_v], row_v, sem).wait()
                pltpu.sync_copy(row_v, dst.at[c].at[pl.ds(b, CHUNK)])

    return gk


def _mlp(x, w1t, b1, w2t, b2, block):
    n, k = x.shape
    h4 = w1t.shape[1]

    def body(x_ref, w1_ref, b1_ref, w2_ref, b2_ref, o_ref):
        h = jnp.dot(x_ref[...], w1_ref[...], preferred_element_type=f32) + b1_ref[...]
        h = jnp.where(h > 0, h, 0.01 * h)
        o_ref[...] = jnp.dot(h, w2_ref[...], preferred_element_type=f32) + b2_ref[...]

    return pl.pallas_call(
        body,
        grid=(n // block,),
        in_specs=[
            pl.BlockSpec((block, k), lambda i: (i, 0)),
            pl.BlockSpec((k, h4), lambda i: (0, 0)),
            pl.BlockSpec((1, h4), lambda i: (0, 0)),
            pl.BlockSpec((h4, D), lambda i: (0, 0)),
            pl.BlockSpec((1, D), lambda i: (0, 0)),
        ],
        out_specs=pl.BlockSpec((block, D), lambda i: (i, 0)),
        out_shape=jax.ShapeDtypeStruct((n, D), f32),
    )(x, w1t, b1, w2t, b2)


def _final(a_i, b_i, a_t, b_t, it_i, it_t, g1wt, g1b, g2wt, g2b, g3wt, g3b, g4wt, g4b):
    def sig(x):
        return 1.0 / (1.0 + jnp.exp(-x))

    def body(ai, bi, at, bt, ii, it, g1, c1, g2, c2, g3, c3, g4, c4, o_ref):
        av, bv = ai[...], bi[...]
        gate = sig(jnp.dot(av, g1[...], preferred_element_type=f32) + c1[...]
                   + jnp.dot(bv, g2[...], preferred_element_type=f32) + c2[...])
        uf_i = gate * av + (1.0 - gate) * bv
        av, bv = at[...], bt[...]
        gate = sig(jnp.dot(av, g3[...], preferred_element_type=f32) + c3[...]
                   + jnp.dot(bv, g4[...], preferred_element_type=f32) + c4[...])
        uf_t = gate * av + (1.0 - gate) * bv
        s = jnp.sum(uf_i * ii[...] + uf_t * it[...], axis=1)
        o_ref[...] = sig(s)

    return pl.pallas_call(
        body,
        out_shape=jax.ShapeDtypeStruct((B,), f32),
    )(a_i, b_i, a_t, b_t, it_i, it_t, g1wt, g1b, g2wt, g2b, g3wt, g3b, g4wt, g4b)


def _pad_edges(head, tail, etype, dump_row, group):
    e = head.shape[0]
    e_pad = -(-e // group) * group
    pad = e_pad - e
    head = jnp.concatenate([head, jnp.full((pad,), dump_row, i32)])
    tail = jnp.concatenate([tail, jnp.zeros((pad,), i32)])
    etype = jnp.concatenate([etype, jnp.zeros((pad,), i32)])
    return head, tail, etype, e_pad


def _halves(x):
    n = x.shape[0]
    return x.reshape(n, 2, H).transpose(1, 0, 2)


def _unhalves(x3):
    return x3.transpose(1, 0, 2).reshape(x3.shape[1], D)


def kernel(user_ids, item_ids, edge_index, edge_type, ukg_edge_index, ukg_edge_type,
           image_features, text_features,
           W_img1, b_img1, W_img2, b_img2, W_txt1, b_txt1, W_txt2, b_txt2,
           other_emb_image, other_emb_text, rel_emb_image, rel_emb_text,
           ukg_rel_emb_image, ukg_rel_emb_text,
           g1W, g1b, g2W, g2b, g3W, g3b, g4W, g4b):
    group = NSUB * CHUNK
    head = jnp.asarray(edge_index[0], i32)
    tail = jnp.asarray(edge_index[1], i32)
    et = jnp.asarray(edge_type, i32)
    uhead = jnp.asarray(ukg_edge_index[0], i32)
    utail = jnp.asarray(ukg_edge_index[1], i32)
    uet = jnp.asarray(ukg_edge_type, i32)

    head, tail, et, e_pad = _pad_edges(head, tail, et, N_NODES, group)
    uhead, utail, uet, ue_pad = _pad_edges(uhead, utail, uet, N_USERS, group)

    n_pad = 51200
    u_pad = 10240

    img = _mlp(image_features, W_img1.T, b_img1[None, :], W_img2.T, b_img2[None, :], 2000)
    txt = _mlp(text_features, W_txt1.T, b_txt1[None, :], W_txt2.T, b_txt2[None, :], 2000)

    zpad = jnp.zeros((2, n_pad - N_NODES, H), f32)
    ego_i3 = jnp.concatenate([_halves(img), _halves(other_emb_image), zpad], axis=1)
    ego_t3 = jnp.concatenate([_halves(txt), _halves(other_emb_text), zpad], axis=1)
    rel_i3 = _halves(rel_emb_image)
    rel_t3 = _halves(rel_emb_text)
    urel_i3 = _halves(ukg_rel_emb_image)
    urel_t3 = _halves(ukg_rel_emb_text)

    recip_kg = _make_deg_kernel(n_pad, e_pad)(head)
    recip_ukg = _make_deg_kernel(u_pad, ue_pad)(uhead)

    hop1 = _make_hop_kernel(n_pad, e_pad, False)
    hop2 = _make_hop_kernel(n_pad, e_pad, True)
    agg1_i = hop1(ego_i3, rel_i3, head, tail, et, recip_kg)
    _, out_i3 = hop2(agg1_i, rel_i3, head, tail, et, recip_kg, ego_i3, agg1_i)
    agg1_t = hop1(ego_t3, rel_t3, head, tail, et, recip_kg)
    _, out_t3 = hop2(agg1_t, rel_t3, head, tail, et, recip_kg, ego_t3, agg1_t)

    uzpad = jnp.zeros((2, u_pad - N_USERS, H), f32)
    u_ego_i3 = jnp.concatenate(
        [out_i3[:, N_ENTITIES:N_NODES, :], uzpad], axis=1)
    u_ego_t3 = jnp.concatenate(
        [out_t3[:, N_ENTITIES:N_NODES, :], uzpad], axis=1)
    uhop1 = _make_hop_kernel(u_pad, ue_pad, False)
    uhop2 = _make_hop_kernel(u_pad, ue_pad, True)
    uagg1_i = uhop1(u_ego_i3, urel_i3, uhead, utail, uet, recip_ukg)
    _, fu_i3 = uhop2(uagg1_i, urel_i3, uhead, utail, uet, recip_ukg, u_ego_i3, uagg1_i)
    uagg1_t = uhop1(u_ego_t3, urel_t3, uhead, utail, uet, recip_ukg)
    _, fu_t3 = uhop2(uagg1_t, urel_t3, uhead, utail, uet, recip_ukg, u_ego_t3, uagg1_t)

    uid = jnp.asarray(user_ids, i32)
    ul = uid - N_ENTITIES
    iid = jnp.asarray(item_ids, i32)
    ai3, bi3, at3, bt3, ii3, it3 = _make_gather_kernel()(
        out_i3, out_t3, fu_i3, fu_t3, ul, uid, iid)

    return _final(_unhalves(ai3), _unhalves(bi3), _unhalves(at3), _unhalves(bt3),
                  _unhalves(ii3), _unhalves(it3),
                  g1W.T, g1b[None, :], g2W.T, g2b[None, :],
                  g3W.T, g3b[None, :], g4W.T, g4b[None, :])
